# baseline (device time: 16869 ns/iter reference)
import os

import jax
import jax.numpy as jnp
from jax import lax
from jax.experimental import pallas as pl
from jax.experimental.pallas import tpu as pltpu

N_DEV = 16
_MODE = os.environ.get("KMODE", "full")
if os.environ.get("SKIP_RDMA") == "1":
    _MODE = "compute"


def kernel(x, W, labels):
    T, D = x.shape
    V_LOC = W.shape[1]

    def body(x_ref, w_ref, lab_ref, out_ref, comm_ref, send_sems, recv_sems):
        my = lax.axis_index("i")

        logits = jnp.dot(
            x_ref[:, :].astype(jnp.bfloat16),
            w_ref[:, :].astype(jnp.bfloat16),
            preferred_element_type=jnp.float32,
        )
        m = jnp.max(logits, axis=1)
        s = jnp.sum(jnp.exp(logits - m[:, None]), axis=1)
        local_tgt = lab_ref[:] - my * V_LOC
        col = lax.broadcasted_iota(jnp.int32, (T, V_LOC), 1)
        lab = jnp.sum(
            jnp.where(col == local_tgt[:, None], logits, 0.0), axis=1
        )

        comm_ref[0, 0, :] = m
        comm_ref[0, 1, :] = s
        comm_ref[0, 2, :] = lab

        if _MODE == "explicit":
            barrier_sem = pltpu.get_barrier_semaphore()
            for d in range(1, N_DEV):
                pl.semaphore_signal(
                    barrier_sem,
                    inc=1,
                    device_id=(lax.rem(my + d, N_DEV),),
                    device_id_type=pl.DeviceIdType.MESH,
                )
            pl.semaphore_wait(barrier_sem, N_DEV - 1)

        if _MODE != "compute":
            rdmas = []
            for d in range(1, N_DEV):
                tgt = lax.rem(my + d, N_DEV)
                rdma = pltpu.make_async_remote_copy(
                    src_ref=comm_ref.at[0],
                    dst_ref=comm_ref.at[d],
                    send_sem=send_sems.at[d],
                    recv_sem=recv_sems.at[d],
                    device_id=(tgt,),
                    device_id_type=pl.DeviceIdType.MESH,
                )
                rdma.start()
                rdmas.append(rdma)
            if _MODE != "nowait":
                for rdma in rdmas:
                    rdma.wait()

        allm = comm_ref[:, 0, :]
        alls = comm_ref[:, 1, :]
        alllab = comm_ref[:, 2, :]
        M = jnp.max(allm, axis=0)
        Z = jnp.sum(alls * jnp.exp(allm - M[None, :]), axis=0)
        lab_tot = jnp.sum(alllab, axis=0)
        out_ref[:] = M + jnp.log(Z) - lab_tot

    return pl.pallas_call(
        body,
        out_shape=jax.ShapeDtypeStruct((T,), jnp.float32),
        in_specs=[
            pl.BlockSpec(memory_space=pltpu.VMEM),
            pl.BlockSpec(memory_space=pltpu.VMEM),
            pl.BlockSpec(memory_space=pltpu.VMEM),
        ],
        out_specs=pl.BlockSpec(memory_space=pltpu.VMEM),
        scratch_shapes=[
            pltpu.VMEM((N_DEV, 3, T), jnp.float32),
            pltpu.SemaphoreType.DMA((N_DEV,)),
            pltpu.SemaphoreType.DMA((N_DEV,)),
        ],
        **(
            dict(compiler_params=pltpu.CompilerParams(collective_id=0))
            if _MODE == "explicit"
            else {}
        ),
    )(x, W, labels)


# device time: 16279 ns/iter; 1.0362x vs baseline; 1.0362x over previous
import os

import jax
import jax.numpy as jnp
from jax import lax
from jax.experimental import pallas as pl
from jax.experimental.pallas import tpu as pltpu

N_DEV = 16
_MODE = os.environ.get("KMODE", "full")


def kernel(x, W, labels):
    T, D = x.shape
    V_LOC = W.shape[1]

    def body(x_ref, w_ref, lab_ref, out_ref, comm_ref, send_sems, recv_sems):
        my = lax.axis_index("i")

        if _MODE != "compute":
            barrier_sem = pltpu.get_barrier_semaphore()
            for d in range(1, N_DEV):
                pl.semaphore_signal(
                    barrier_sem,
                    inc=1,
                    device_id=(lax.rem(my + d, N_DEV),),
                    device_id_type=pl.DeviceIdType.MESH,
                )

        logits = jnp.dot(
            x_ref[:, :].astype(jnp.bfloat16),
            w_ref[:, :].astype(jnp.bfloat16),
            preferred_element_type=jnp.float32,
        ).astype(jnp.bfloat16)
        m = jnp.max(logits, axis=1)
        s = jnp.sum(
            jnp.exp(logits - m[:, None]), axis=1, dtype=jnp.float32
        )
        local_tgt = lab_ref[:] - my * V_LOC
        col = lax.broadcasted_iota(jnp.int32, (T, V_LOC), 1)
        lab = jnp.sum(
            jnp.where(col == local_tgt[:, None], logits, jnp.bfloat16(0.0)),
            axis=1,
            dtype=jnp.float32,
        )

        comm_ref[0, 0, :] = m.astype(jnp.float32)
        comm_ref[0, 1, :] = s
        comm_ref[0, 2, :] = lab

        if _MODE != "compute":
            pl.semaphore_wait(barrier_sem, N_DEV - 1)

            rdmas = []
            for d in range(1, N_DEV):
                rdma = pltpu.make_async_remote_copy(
                    src_ref=comm_ref.at[0],
                    dst_ref=comm_ref.at[d],
                    send_sem=send_sems.at[d],
                    recv_sem=recv_sems.at[d],
                    device_id=(lax.rem(my + d, N_DEV),),
                    device_id_type=pl.DeviceIdType.MESH,
                )
                rdma.start()
                rdmas.append(rdma)
            for rdma in rdmas:
                rdma.wait()

        allm = comm_ref[:, 0, :]
        alls = comm_ref[:, 1, :]
        alllab = comm_ref[:, 2, :]
        M = jnp.max(allm, axis=0)
        Z = jnp.sum(alls * jnp.exp(allm - M[None, :]), axis=0)
        lab_tot = jnp.sum(alllab, axis=0)
        out_ref[:] = M + jnp.log(Z) - lab_tot

    return pl.pallas_call(
        body,
        out_shape=jax.ShapeDtypeStruct((T,), jnp.float32),
        in_specs=[
            pl.BlockSpec(memory_space=pltpu.VMEM),
            pl.BlockSpec(memory_space=pltpu.VMEM),
            pl.BlockSpec(memory_space=pltpu.VMEM),
        ],
        out_specs=pl.BlockSpec(memory_space=pltpu.VMEM),
        scratch_shapes=[
            pltpu.VMEM((N_DEV, 3, T), jnp.float32),
            pltpu.SemaphoreType.DMA((N_DEV,)),
            pltpu.SemaphoreType.DMA((N_DEV,)),
        ],
        **(
            {}
            if _MODE == "compute"
            else dict(compiler_params=pltpu.CompilerParams(collective_id=0))
        ),
    )(x, W, labels)
